# baseline (device time: 34550 ns/iter reference)
import jax
import jax.numpy as jnp
from jax import lax
from jax.experimental import pallas as pl
from jax.experimental.pallas import tpu as pltpu

N_DEV = 8
BLK = 256


def _gelu(y):
    c = 0.7978845608028654
    return 0.5 * y * (1.0 + jnp.tanh(c * (y + 0.044715 * y * y * y)))


def kernel(x, w_mat):
    k, m_per = x.shape
    kw, n = w_mat.shape

    def body(x_ref, w_ref, out_ref, gather_ref, send_sems, recv_sems):
        my = lax.axis_index("i")

        rdmas = []
        for d in range(1, N_DEV):
            dst = lax.rem(my + d, N_DEV)
            rdma = pltpu.make_async_remote_copy(
                src_ref=x_ref.at[pl.ds(dst * BLK, BLK), :],
                dst_ref=gather_ref.at[N_DEV - d],
                send_sem=send_sems.at[d],
                recv_sem=recv_sems.at[N_DEV - d],
                device_id=(dst,),
                device_id_type=pl.DeviceIdType.MESH,
            )
            rdma.start()
            rdmas.append(rdma)

        xloc = x_ref[pl.ds(my * BLK, BLK), :]
        wloc = w_ref[pl.ds(my * BLK, BLK), :]
        out_ref[:, :] = jnp.dot(xloc, wloc, preferred_element_type=jnp.float32)

        for q in range(1, N_DEV):
            rdmas[N_DEV - 1 - q].wait_recv()
            src = lax.rem(my + q, N_DEV)
            wblk = w_ref[pl.ds(src * BLK, BLK), :]
            out_ref[:, :] += jnp.dot(
                gather_ref[q], wblk, preferred_element_type=jnp.float32
            )

        out_ref[:, :] = _gelu(out_ref[:, :])

        for r in rdmas:
            r.wait_send()

    return pl.pallas_call(
        body,
        out_shape=jax.ShapeDtypeStruct((m_per, n), jnp.float32),
        in_specs=[
            pl.BlockSpec(memory_space=pltpu.VMEM),
            pl.BlockSpec(memory_space=pltpu.VMEM),
        ],
        out_specs=pl.BlockSpec(memory_space=pltpu.VMEM),
        scratch_shapes=[
            pltpu.VMEM((N_DEV, BLK, m_per), jnp.float32),
            pltpu.SemaphoreType.DMA((N_DEV,)),
            pltpu.SemaphoreType.DMA((N_DEV,)),
        ],
    )(x, w_mat)


# device time: 30484 ns/iter; 1.1334x vs baseline; 1.1334x over previous
import jax
import jax.numpy as jnp
from jax import lax
from jax.experimental import pallas as pl
from jax.experimental.pallas import tpu as pltpu

N_DEV = 8
BLK = 256


def _gelu(y):
    c = 0.7978845608028654
    return 0.5 * y * (1.0 + jnp.tanh(c * (y + 0.044715 * y * y * y)))


def _decode(i):
    z = i // 4
    p = i % 4
    y = (p >= 2).astype(jnp.int32)
    x = ((p == 1) | (p == 2)).astype(jnp.int32)
    return x, y, z


def _encode(x, y, z):
    return z * 4 + (y * 3 + x - 2 * x * y)


def _neighbor(i, dx, dy, dz):
    x, y, z = _decode(i)
    return _encode(x ^ dx, y ^ dy, z ^ dz)


_MASKS = (
    (1, 0, 0),
    (0, 1, 0),
    (0, 0, 1),
    (1, 1, 0),
    (1, 0, 1),
    (0, 1, 1),
    (1, 1, 1),
)
_WAIT_ORDER = (0, 1, 2, 3, 5, 4, 6)


def kernel(x, w_mat):
    k, m_per = x.shape
    kw, n = w_mat.shape

    def body(x_ref, w_ref, out_ref, gather_ref, send_sems, recv_sems):
        my = lax.axis_index("i")

        bar = pltpu.get_barrier_semaphore()
        for d in range(1, N_DEV):
            peer = lax.rem(my + d, N_DEV)
            pl.semaphore_signal(
                bar, inc=1, device_id=(peer,),
                device_id_type=pl.DeviceIdType.MESH,
            )

        xloc = x_ref[pl.ds(my * BLK, BLK), :]
        wloc = w_ref[pl.ds(my * BLK, BLK), :]
        out_ref[:, :] = jnp.dot(xloc, wloc, preferred_element_type=jnp.float32)

        pl.semaphore_wait(bar, N_DEV - 1)

        rdmas = []
        for j, (dx, dy, dz) in enumerate(_MASKS):
            dst = _neighbor(my, dx, dy, dz)
            rdma = pltpu.make_async_remote_copy(
                src_ref=x_ref.at[pl.ds(dst * BLK, BLK), :],
                dst_ref=gather_ref.at[j],
                send_sem=send_sems.at[j],
                recv_sem=recv_sems.at[j],
                device_id=(dst,),
                device_id_type=pl.DeviceIdType.MESH,
            )
            rdma.start()
            rdmas.append(rdma)

        for j in _WAIT_ORDER:
            rdmas[j].wait_recv()
            dx, dy, dz = _MASKS[j]
            src = _neighbor(my, dx, dy, dz)
            wblk = w_ref[pl.ds(src * BLK, BLK), :]
            out_ref[:, :] += jnp.dot(
                gather_ref[j], wblk, preferred_element_type=jnp.float32
            )

        out_ref[:, :] = _gelu(out_ref[:, :])

        for r in rdmas:
            r.wait_send()

    return pl.pallas_call(
        body,
        out_shape=jax.ShapeDtypeStruct((m_per, n), jnp.float32),
        in_specs=[
            pl.BlockSpec(memory_space=pltpu.VMEM),
            pl.BlockSpec(memory_space=pltpu.VMEM),
        ],
        out_specs=pl.BlockSpec(memory_space=pltpu.VMEM),
        scratch_shapes=[
            pltpu.VMEM((N_DEV - 1, BLK, m_per), jnp.float32),
            pltpu.SemaphoreType.DMA((N_DEV - 1,)),
            pltpu.SemaphoreType.DMA((N_DEV - 1,)),
        ],
        compiler_params=pltpu.CompilerParams(collective_id=0),
    )(x, w_mat)


# device time: 29744 ns/iter; 1.1616x vs baseline; 1.0249x over previous
import jax
import jax.numpy as jnp
from jax import lax
from jax.experimental import pallas as pl
from jax.experimental.pallas import tpu as pltpu

N_DEV = 8
BLK = 256
S = 4
PIECE = BLK // S


def _gelu(y):
    c = 0.7978845608028654
    return 0.5 * y * (1.0 + jnp.tanh(c * (y + 0.044715 * y * y * y)))


def _decode(i):
    z = i // 4
    p = i % 4
    y = (p >= 2).astype(jnp.int32)
    x = ((p == 1) | (p == 2)).astype(jnp.int32)
    return x, y, z


def _encode(x, y, z):
    return z * 4 + (y * 3 + x - 2 * x * y)


def _neighbor(i, dx, dy, dz):
    x, y, z = _decode(i)
    return _encode(x ^ dx, y ^ dy, z ^ dz)


_MASKS = (
    (1, 0, 0),
    (0, 1, 0),
    (0, 0, 1),
    (1, 1, 0),
    (1, 0, 1),
    (0, 1, 1),
    (1, 1, 1),
)
_WAIT_ORDER = (0, 1, 2, 3, 5, 4, 6)


def kernel(x, w_mat):
    k, m_per = x.shape
    kw, n = w_mat.shape

    def body(x_ref, w_hbm, out_ref, w_ref, gather_ref,
             send_sems, recv_sems, w_sem):
        my = lax.axis_index("i")

        wcopy = pltpu.make_async_copy(w_hbm, w_ref, w_sem)
        wcopy.start()

        bar = pltpu.get_barrier_semaphore()
        for d in range(1, N_DEV):
            peer = lax.rem(my + d, N_DEV)
            pl.semaphore_signal(
                bar, inc=1, device_id=(peer,),
                device_id_type=pl.DeviceIdType.MESH,
            )
        pl.semaphore_wait(bar, N_DEV - 1)

        rdmas = {}
        for p in range(S):
            for j in range(N_DEV - 1):
                dx, dy, dz = _MASKS[j]
                dst = _neighbor(my, dx, dy, dz)
                r = pltpu.make_async_remote_copy(
                    src_ref=x_ref.at[pl.ds(dst * BLK + p * PIECE, PIECE), :],
                    dst_ref=gather_ref.at[j, pl.ds(p * PIECE, PIECE), :],
                    send_sem=send_sems.at[p, j],
                    recv_sem=recv_sems.at[p, j],
                    device_id=(dst,),
                    device_id_type=pl.DeviceIdType.MESH,
                )
                r.start()
                rdmas[(p, j)] = r

        wcopy.wait()
        xloc = x_ref[pl.ds(my * BLK, BLK), :]
        wloc = w_ref[pl.ds(my * BLK, BLK), :]
        out_ref[:, :] = jnp.dot(xloc, wloc, preferred_element_type=jnp.float32)

        for p in range(S):
            for j in _WAIT_ORDER:
                rdmas[(p, j)].wait_recv()
                dx, dy, dz = _MASKS[j]
                src = _neighbor(my, dx, dy, dz)
                wblk = w_ref[pl.ds(src * BLK, BLK), :]
                out_ref[pl.ds(p * PIECE, PIECE), :] += jnp.dot(
                    gather_ref[j, pl.ds(p * PIECE, PIECE), :],
                    wblk,
                    preferred_element_type=jnp.float32,
                )

        out_ref[:, :] = _gelu(out_ref[:, :])

        for r in rdmas.values():
            r.wait_send()

    return pl.pallas_call(
        body,
        out_shape=jax.ShapeDtypeStruct((m_per, n), jnp.float32),
        in_specs=[
            pl.BlockSpec(memory_space=pltpu.VMEM),
            pl.BlockSpec(memory_space=pl.ANY),
        ],
        out_specs=pl.BlockSpec(memory_space=pltpu.VMEM),
        scratch_shapes=[
            pltpu.VMEM((kw, n), jnp.float32),
            pltpu.VMEM((N_DEV - 1, BLK, m_per), jnp.float32),
            pltpu.SemaphoreType.DMA((S, N_DEV - 1)),
            pltpu.SemaphoreType.DMA((S, N_DEV - 1)),
            pltpu.SemaphoreType.DMA,
        ],
        compiler_params=pltpu.CompilerParams(collective_id=0),
    )(x, w_mat)
